# Initial kernel scaffold; baseline (speedup 1.0000x reference)
import jax
import jax.numpy as jnp
from jax.experimental import pallas as pl


def kernel(x, edge_index, pseudo, params):
    raise NotImplementedError("write your pallas kernel here")


# speedup vs baseline: 8.8555x; 8.8555x over previous
import functools

import jax
import jax.numpy as jnp
from jax import lax
from jax.experimental import pallas as pl
from jax.experimental.pallas import tpu as pltpu
from jax.experimental.pallas import tpu_sc as plsc

NC = 2
NS = 16
NW = NC * NS
C = 128
LANES = 16
G = 128


def _row_chunks(N):
  grans = N // 8
  gpt = (grans // NS) * 8
  gtail = grans - (grans // NS) * NS
  return gpt, gtail


def _edge_kernel(N, E, K, cout):
  chunks = E // C
  nfull = chunks // NW
  rem = chunks - nfull * NW
  gpt, gtail = _row_chunks(N)
  jpr = cout // LANES
  mesh = plsc.VectorSubcoreMesh(core_axis_name="c", subcore_axis_name="s")

  @functools.partial(
      pl.kernel,
      mesh=mesh,
      out_type=jax.ShapeDtypeStruct((2, N, G), jnp.float32),
      scratch_types=[
          pltpu.VMEM((C,), jnp.int32),
          pltpu.VMEM((C,), jnp.int32),
          pltpu.VMEM((C,), jnp.float32),
          pltpu.VMEM((C,), jnp.int32),
          pltpu.VMEM((C,), jnp.int32),
          pltpu.VMEM((C,), jnp.int32),
          pltpu.VMEM((C,), jnp.int32),
          pltpu.VMEM((C,), jnp.float32),
          pltpu.VMEM((C, G), jnp.float32),
          pltpu.VMEM((C, G), jnp.float32),
          pltpu.VMEM((C, G), jnp.float32),
          pltpu.VMEM_SHARED((N, G), jnp.float32),
          pltpu.SemaphoreType.DMA,
          pltpu.SemaphoreType.DMA,
      ],
  )
  def body(zf_hbm, src_hbm, dst_hbm, ps_hbm, zeros_hbm, out_hbm,
           src_v, dst_v, ps_v, g0_v, g1_v, o0_v, o1_v, f_v, r0, r1, msg,
           acc, sem0, sem1):
    cid = lax.axis_index("c")
    sid = lax.axis_index("s")
    wid = cid * NS + sid

    pltpu.sync_copy(zeros_hbm.at[pl.ds(sid * gpt, gpt)],
                    acc.at[pl.ds(sid * gpt, gpt)])
    if gtail:
      @pl.when(sid < gtail)
      def _():
        t0 = NS * gpt + sid * 8
        pltpu.sync_copy(zeros_hbm.at[pl.ds(t0, 8)], acc.at[pl.ds(t0, 8)])

    if jpr < G // LANES:
      def zero_msg(e, carry):
        for j in range(jpr, G // LANES):
          msg[e, pl.ds(j * LANES, LANES)] = jnp.zeros((LANES,), jnp.float32)
        return carry
      lax.fori_loop(0, C, zero_msg, 0)
    plsc.subcore_barrier()

    def do_chunk(c_idx):
      base = c_idx * C
      pltpu.sync_copy(src_hbm.at[pl.ds(base, C)], src_v)
      pltpu.sync_copy(dst_hbm.at[pl.ds(base, C)], dst_v)
      pltpu.sync_copy(ps_hbm.at[pl.ds(base, C)], ps_v)
      for j in range(C // LANES):
        sl = pl.ds(j * LANES, LANES)
        v = ps_v[sl] * jnp.float32(K - 1)
        i0 = v.astype(jnp.int32)
        f_v[sl] = v - i0.astype(jnp.float32)
        flat0 = (src_v[sl] * K + i0) * cout
        flat1 = flat0 + cout
        g0_v[sl] = lax.shift_right_logical(flat0, 7)
        o0_v[sl] = lax.bitwise_and(flat0, 127)
        g1_v[sl] = lax.shift_right_logical(flat1, 7)
        o1_v[sl] = lax.bitwise_and(flat1, 127)
      cp0 = pltpu.async_copy(zf_hbm.at[g0_v], r0, sem0)
      cp1 = pltpu.async_copy(zf_hbm.at[g1_v], r1, sem1)
      cp0.wait()
      cp1.wait()

      def interp(t, carry):
        sl = pl.ds(t * LANES, LANES)
        fg = f_v[sl]
        a0 = o0_v[sl]
        a1 = o1_v[sl]
        for l in range(LANES):
          fb = jnp.full((LANES,), fg[l], jnp.float32)
          o0s = a0[l]
          o1s = a1[l]
          e = t * LANES + l
          for j in range(jpr):
            a = r0[e, pl.ds(o0s + j * LANES, LANES)]
            b = r1[e, pl.ds(o1s + j * LANES, LANES)]
            msg[e, pl.ds(j * LANES, LANES)] = a + fb * (b - a)
        return carry

      lax.fori_loop(0, C // LANES, interp, 0)
      pltpu.sync_copy(msg, acc.at[dst_v], add=True)

    def chunk_loop(t, carry):
      do_chunk(wid + NW * t)
      return carry

    lax.fori_loop(0, nfull, chunk_loop, 0)
    if rem:
      @pl.when(wid < rem)
      def _():
        do_chunk(NW * nfull + wid)

    plsc.subcore_barrier()
    pltpu.sync_copy(acc.at[pl.ds(sid * gpt, gpt)],
                    out_hbm.at[cid, pl.ds(sid * gpt, gpt)])
    if gtail:
      @pl.when(sid < gtail)
      def _():
        t0 = NS * gpt + sid * 8
        pltpu.sync_copy(acc.at[pl.ds(t0, 8)], out_hbm.at[cid, pl.ds(t0, 8)])

  return body


def _deg_kernel(N, E):
  chunks = E // C
  nfull = chunks // NW
  rem = chunks - nfull * NW
  gpt, gtail = _row_chunks(N)
  mesh = plsc.VectorSubcoreMesh(core_axis_name="c", subcore_axis_name="s")

  @functools.partial(
      pl.kernel,
      mesh=mesh,
      out_type=jax.ShapeDtypeStruct((2, N, G), jnp.float32),
      scratch_types=[
          pltpu.VMEM((C,), jnp.int32),
          pltpu.VMEM((C, G), jnp.float32),
          pltpu.VMEM_SHARED((N, G), jnp.float32),
      ],
  )
  def body(dst_hbm, zeros_hbm, ones_hbm, out_hbm, dst_v, ones_v, acc):
    cid = lax.axis_index("c")
    sid = lax.axis_index("s")
    wid = cid * NS + sid

    pltpu.sync_copy(ones_hbm, ones_v)
    pltpu.sync_copy(zeros_hbm.at[pl.ds(sid * gpt, gpt)],
                    acc.at[pl.ds(sid * gpt, gpt)])
    if gtail:
      @pl.when(sid < gtail)
      def _():
        t0 = NS * gpt + sid * 8
        pltpu.sync_copy(zeros_hbm.at[pl.ds(t0, 8)], acc.at[pl.ds(t0, 8)])
    plsc.subcore_barrier()

    def do_chunk(c_idx):
      pltpu.sync_copy(dst_hbm.at[pl.ds(c_idx * C, C)], dst_v)
      pltpu.sync_copy(ones_v, acc.at[dst_v], add=True)

    def chunk_loop(t, carry):
      do_chunk(wid + NW * t)
      return carry

    lax.fori_loop(0, nfull, chunk_loop, 0)
    if rem:
      @pl.when(wid < rem)
      def _():
        do_chunk(NW * nfull + wid)

    plsc.subcore_barrier()
    pltpu.sync_copy(acc.at[pl.ds(sid * gpt, gpt)],
                    out_hbm.at[cid, pl.ds(sid * gpt, gpt)])
    if gtail:
      @pl.when(sid < gtail)
      def _():
        t0 = NS * gpt + sid * 8
        pltpu.sync_copy(acc.at[pl.ds(t0, 8)], out_hbm.at[cid, pl.ds(t0, 8)])

  return body


_BR = 1000


def _mm0_body(x_ref, w_ref, o1_ref, o2_ref, *, kc):
  z = jnp.dot(x_ref[...], w_ref[...], preferred_element_type=jnp.float32)
  o1_ref[...] = z[:, :kc]
  o2_ref[...] = z[:, kc:]


def _epilogue(p_ref, dp_ref, rt_ref, b_ref, cp, elu):
  d = dp_ref[0, :, 0:1] + dp_ref[1, :, 0:1]
  inv = 1.0 / jnp.maximum(d, 1.0)
  h = (p_ref[0, :, :cp] + p_ref[1, :, :cp]) * inv + rt_ref[...] + b_ref[...]
  if elu:
    h = jnp.where(h > 0, h, jnp.exp(jnp.minimum(h, 0.0)) - 1.0)
  return h


def _mm_fused_body(p_ref, dp_ref, rt_ref, b_ref, w_ref, o1_ref, o2_ref,
                   *, cp, kc, elu):
  h = _epilogue(p_ref, dp_ref, rt_ref, b_ref, cp, elu)
  z = jnp.dot(h, w_ref[...], preferred_element_type=jnp.float32)
  o1_ref[...] = z[:, :kc]
  o2_ref[...] = z[:, kc:]


def _final_body(p_ref, dp_ref, rt_ref, b_ref, o_ref, *, cp):
  h = _epilogue(p_ref, dp_ref, rt_ref, b_ref, cp, False)
  m = jnp.max(h, axis=1, keepdims=True)
  l = h - m
  s = jnp.sum(jnp.exp(l), axis=1, keepdims=True)
  o_ref[...] = l - jnp.log(s)


def _mm0(N, cin, kc, cout):
  cols = kc + cout
  return pl.pallas_call(
      functools.partial(_mm0_body, kc=kc),
      grid=(N // _BR,),
      in_specs=[
          pl.BlockSpec((_BR, cin), lambda i: (i, 0)),
          pl.BlockSpec((cin, cols), lambda i: (0, 0)),
      ],
      out_specs=[
          pl.BlockSpec((_BR, kc), lambda i: (i, 0)),
          pl.BlockSpec((_BR, cout), lambda i: (i, 0)),
      ],
      out_shape=[
          jax.ShapeDtypeStruct((N, kc), jnp.float32),
          jax.ShapeDtypeStruct((N, cout), jnp.float32),
      ],
  )


def _mm_fused(N, cp, kc, cout, elu):
  cols = kc + cout
  return pl.pallas_call(
      functools.partial(_mm_fused_body, cp=cp, kc=kc, elu=elu),
      grid=(N // _BR,),
      in_specs=[
          pl.BlockSpec((2, _BR, G), lambda i: (0, i, 0)),
          pl.BlockSpec((2, _BR, G), lambda i: (0, i, 0)),
          pl.BlockSpec((_BR, cp), lambda i: (i, 0)),
          pl.BlockSpec((1, cp), lambda i: (0, 0)),
          pl.BlockSpec((cp, cols), lambda i: (0, 0)),
      ],
      out_specs=[
          pl.BlockSpec((_BR, kc), lambda i: (i, 0)),
          pl.BlockSpec((_BR, cout), lambda i: (i, 0)),
      ],
      out_shape=[
          jax.ShapeDtypeStruct((N, kc), jnp.float32),
          jax.ShapeDtypeStruct((N, cout), jnp.float32),
      ],
  )


def _final(N, cp):
  return pl.pallas_call(
      functools.partial(_final_body, cp=cp),
      grid=(N // _BR,),
      in_specs=[
          pl.BlockSpec((2, _BR, G), lambda i: (0, i, 0)),
          pl.BlockSpec((2, _BR, G), lambda i: (0, i, 0)),
          pl.BlockSpec((_BR, cp), lambda i: (i, 0)),
          pl.BlockSpec((1, cp), lambda i: (0, 0)),
      ],
      out_specs=pl.BlockSpec((_BR, cp), lambda i: (i, 0)),
      out_shape=jax.ShapeDtypeStruct((N, cp), jnp.float32),
  )


@jax.jit
def kernel(x, edge_index, pseudo, params):
  N = x.shape[0]
  E = edge_index.shape[1]
  src = edge_index[0]
  dst = edge_index[1]
  ps = pseudo[:, 0]
  zeros_ng = jnp.zeros((N, G), jnp.float32)

  deg_parts = _deg_kernel(N, E)(
      dst, zeros_ng, jnp.ones((C, G), jnp.float32))

  layer_dims = [(p["weight"].shape[1], p["weight"].shape[2], p["weight"].shape[0])
                for p in params]
  wcats = []
  for p, (cin, cout, K) in zip(params, layer_dims):
    wflat = jnp.transpose(p["weight"], (1, 0, 2)).reshape(cin, K * cout)
    wcats.append(jnp.concatenate([wflat, p["root"]], axis=1))

  cin0, cout0, k0 = layer_dims[0]
  z, rt = _mm0(N, cin0, k0 * cout0, cout0)(x, wcats[0])

  for li, (cin, cout, K) in enumerate(layer_dims):
    parts = _edge_kernel(N, E, K, cout)(
        z.reshape(N * K * cout // G, G), src, dst, ps, zeros_ng)
    bias = params[li]["bias"].reshape(1, cout)
    if li + 1 < len(layer_dims):
      cin_n, cout_n, k_n = layer_dims[li + 1]
      z, rt = _mm_fused(N, cout, k_n * cout_n, cout_n, elu=li in (0, 2, 4))(
          parts, deg_parts, rt, bias, wcats[li + 1])
    else:
      return _final(N, cout)(parts, deg_parts, rt, bias)


# speedup vs baseline: 10.2642x; 1.1591x over previous
import functools

import jax
import jax.numpy as jnp
from jax import lax
from jax.experimental import pallas as pl
from jax.experimental.pallas import tpu as pltpu
from jax.experimental.pallas import tpu_sc as plsc

NC = 2
NS = 16
NW = NC * NS
C = 128
LANES = 16
G = 128


def _row_chunks(N):
  grans = N // 8
  gpt = (grans // NS) * 8
  gtail = grans - (grans // NS) * NS
  return gpt, gtail


def _edge_kernel(N, E, K, cout):
  chunks = E // C
  nfull = chunks // NW
  rem = chunks - nfull * NW
  gpt, gtail = _row_chunks(N)
  mesh = plsc.VectorSubcoreMesh(core_axis_name="c", subcore_axis_name="s")

  @functools.partial(
      pl.kernel,
      mesh=mesh,
      out_type=jax.ShapeDtypeStruct((2, N, G), jnp.float32),
      scratch_types=[
          pltpu.VMEM((C,), jnp.int32),
          pltpu.VMEM((C,), jnp.int32),
          pltpu.VMEM((C,), jnp.float32),
          pltpu.VMEM((C,), jnp.int32),
          pltpu.VMEM((C,), jnp.int32),
          pltpu.VMEM((C,), jnp.float32),
          pltpu.VMEM((C, G), jnp.float32),
          pltpu.VMEM((C, G), jnp.float32),
          pltpu.VMEM_SHARED((N, G), jnp.float32),
          pltpu.SemaphoreType.DMA,
          pltpu.SemaphoreType.DMA,
      ],
  )
  def body(zf_hbm, src_hbm, dst_hbm, ps_hbm, zeros_hbm, out_hbm,
           src_v, dst_v, ps_v, g0_v, g1_v, f_v, r0, r1,
           acc, sem0, sem1):
    cid = lax.axis_index("c")
    sid = lax.axis_index("s")
    wid = cid * NS + sid

    pltpu.sync_copy(zeros_hbm.at[pl.ds(sid * gpt, gpt)],
                    acc.at[pl.ds(sid * gpt, gpt)])
    if gtail:
      @pl.when(sid < gtail)
      def _():
        t0 = NS * gpt + sid * 8
        pltpu.sync_copy(zeros_hbm.at[pl.ds(t0, 8)], acc.at[pl.ds(t0, 8)])

    plsc.subcore_barrier()

    def do_chunk(c_idx):
      base = c_idx * C
      pltpu.sync_copy(src_hbm.at[pl.ds(base, C)], src_v)
      pltpu.sync_copy(dst_hbm.at[pl.ds(base, C)], dst_v)
      pltpu.sync_copy(ps_hbm.at[pl.ds(base, C)], ps_v)
      for j in range(C // LANES):
        sl = pl.ds(j * LANES, LANES)
        v = ps_v[sl] * jnp.float32(K - 1)
        i0 = v.astype(jnp.int32)
        f_v[sl] = v - i0.astype(jnp.float32)
        row0 = src_v[sl] * K + i0
        g0_v[sl] = row0
        g1_v[sl] = row0 + 1
      cp0 = pltpu.async_copy(zf_hbm.at[g0_v], r0, sem0)
      cp1 = pltpu.async_copy(zf_hbm.at[g1_v], r1, sem1)
      cp0.wait()
      cp1.wait()

      jpr = cout // LANES

      def interp(t, carry):
        fg = f_v[pl.ds(t * LANES, LANES)]
        for l in range(LANES):
          fb = jnp.take_along_axis(
              fg, jnp.full((LANES,), l, jnp.int32), axis=0,
              mode="promise_in_bounds")
          e = t * LANES + l
          for j in range(jpr):
            sl = pl.ds(j * LANES, LANES)
            a = r0[e, sl]
            b = r1[e, sl]
            r0[e, sl] = a + fb * (b - a)
        return carry

      lax.fori_loop(0, C // LANES, interp, 0)
      pltpu.sync_copy(r0, acc.at[dst_v], add=True)

    def chunk_loop(t, carry):
      do_chunk(wid + NW * t)
      return carry

    lax.fori_loop(0, nfull, chunk_loop, 0)
    if rem:
      @pl.when(wid < rem)
      def _():
        do_chunk(NW * nfull + wid)

    plsc.subcore_barrier()
    pltpu.sync_copy(acc.at[pl.ds(sid * gpt, gpt)],
                    out_hbm.at[cid, pl.ds(sid * gpt, gpt)])
    if gtail:
      @pl.when(sid < gtail)
      def _():
        t0 = NS * gpt + sid * 8
        pltpu.sync_copy(acc.at[pl.ds(t0, 8)], out_hbm.at[cid, pl.ds(t0, 8)])

  return body


def _deg_kernel(N, E):
  chunks = E // C
  nfull = chunks // NW
  rem = chunks - nfull * NW
  gpt, gtail = _row_chunks(N)
  mesh = plsc.VectorSubcoreMesh(core_axis_name="c", subcore_axis_name="s")

  @functools.partial(
      pl.kernel,
      mesh=mesh,
      out_type=jax.ShapeDtypeStruct((2, N, G), jnp.float32),
      scratch_types=[
          pltpu.VMEM((C,), jnp.int32),
          pltpu.VMEM((C, G), jnp.float32),
          pltpu.VMEM_SHARED((N, G), jnp.float32),
      ],
  )
  def body(dst_hbm, zeros_hbm, ones_hbm, out_hbm, dst_v, ones_v, acc):
    cid = lax.axis_index("c")
    sid = lax.axis_index("s")
    wid = cid * NS + sid

    pltpu.sync_copy(ones_hbm, ones_v)
    pltpu.sync_copy(zeros_hbm.at[pl.ds(sid * gpt, gpt)],
                    acc.at[pl.ds(sid * gpt, gpt)])
    if gtail:
      @pl.when(sid < gtail)
      def _():
        t0 = NS * gpt + sid * 8
        pltpu.sync_copy(zeros_hbm.at[pl.ds(t0, 8)], acc.at[pl.ds(t0, 8)])
    plsc.subcore_barrier()

    def do_chunk(c_idx):
      pltpu.sync_copy(dst_hbm.at[pl.ds(c_idx * C, C)], dst_v)
      pltpu.sync_copy(ones_v, acc.at[dst_v], add=True)

    def chunk_loop(t, carry):
      do_chunk(wid + NW * t)
      return carry

    lax.fori_loop(0, nfull, chunk_loop, 0)
    if rem:
      @pl.when(wid < rem)
      def _():
        do_chunk(NW * nfull + wid)

    plsc.subcore_barrier()
    pltpu.sync_copy(acc.at[pl.ds(sid * gpt, gpt)],
                    out_hbm.at[cid, pl.ds(sid * gpt, gpt)])
    if gtail:
      @pl.when(sid < gtail)
      def _():
        t0 = NS * gpt + sid * 8
        pltpu.sync_copy(acc.at[pl.ds(t0, 8)], out_hbm.at[cid, pl.ds(t0, 8)])

  return body


_BR = 1000


def _pad_table(z, K, cout):
  if cout == G:
    return z
  br = z.shape[0]
  zeros = jnp.zeros((br, G - cout), jnp.float32)
  pieces = []
  for k in range(K):
    pieces.append(z[:, k * cout:(k + 1) * cout])
    pieces.append(zeros)
  return jnp.concatenate(pieces, axis=1)


def _mm0_body(x_ref, w_ref, o1_ref, o2_ref, *, K, cout):
  kc = K * cout
  z = jnp.dot(x_ref[...], w_ref[...], preferred_element_type=jnp.float32)
  o1_ref[...] = _pad_table(z[:, :kc], K, cout)
  o2_ref[...] = z[:, kc:]


def _epilogue(p_ref, dp_ref, rt_ref, b_ref, cp, elu):
  d = dp_ref[0, :, 0:1] + dp_ref[1, :, 0:1]
  inv = 1.0 / jnp.maximum(d, 1.0)
  h = (p_ref[0, :, :cp] + p_ref[1, :, :cp]) * inv + rt_ref[...] + b_ref[...]
  if elu:
    h = jnp.where(h > 0, h, jnp.exp(jnp.minimum(h, 0.0)) - 1.0)
  return h


def _mm_fused_body(p_ref, dp_ref, rt_ref, b_ref, w_ref, o1_ref, o2_ref,
                   *, cp, K, cout, elu):
  kc = K * cout
  h = _epilogue(p_ref, dp_ref, rt_ref, b_ref, cp, elu)
  z = jnp.dot(h, w_ref[...], preferred_element_type=jnp.float32)
  o1_ref[...] = _pad_table(z[:, :kc], K, cout)
  o2_ref[...] = z[:, kc:]


def _final_body(p_ref, dp_ref, rt_ref, b_ref, o_ref, *, cp):
  h = _epilogue(p_ref, dp_ref, rt_ref, b_ref, cp, False)
  m = jnp.max(h, axis=1, keepdims=True)
  l = h - m
  s = jnp.sum(jnp.exp(l), axis=1, keepdims=True)
  o_ref[...] = l - jnp.log(s)


def _mm0(N, cin, K, cout):
  cols = (K + 1) * cout
  return pl.pallas_call(
      functools.partial(_mm0_body, K=K, cout=cout),
      grid=(N // _BR,),
      in_specs=[
          pl.BlockSpec((_BR, cin), lambda i: (i, 0)),
          pl.BlockSpec((cin, cols), lambda i: (0, 0)),
      ],
      out_specs=[
          pl.BlockSpec((_BR, K * G), lambda i: (i, 0)),
          pl.BlockSpec((_BR, cout), lambda i: (i, 0)),
      ],
      out_shape=[
          jax.ShapeDtypeStruct((N, K * G), jnp.float32),
          jax.ShapeDtypeStruct((N, cout), jnp.float32),
      ],
  )


def _mm_fused(N, cp, K, cout, elu):
  cols = (K + 1) * cout
  return pl.pallas_call(
      functools.partial(_mm_fused_body, cp=cp, K=K, cout=cout, elu=elu),
      grid=(N // _BR,),
      in_specs=[
          pl.BlockSpec((2, _BR, G), lambda i: (0, i, 0)),
          pl.BlockSpec((2, _BR, G), lambda i: (0, i, 0)),
          pl.BlockSpec((_BR, cp), lambda i: (i, 0)),
          pl.BlockSpec((1, cp), lambda i: (0, 0)),
          pl.BlockSpec((cp, cols), lambda i: (0, 0)),
      ],
      out_specs=[
          pl.BlockSpec((_BR, K * G), lambda i: (i, 0)),
          pl.BlockSpec((_BR, cout), lambda i: (i, 0)),
      ],
      out_shape=[
          jax.ShapeDtypeStruct((N, K * G), jnp.float32),
          jax.ShapeDtypeStruct((N, cout), jnp.float32),
      ],
  )


def _final(N, cp):
  return pl.pallas_call(
      functools.partial(_final_body, cp=cp),
      grid=(N // _BR,),
      in_specs=[
          pl.BlockSpec((2, _BR, G), lambda i: (0, i, 0)),
          pl.BlockSpec((2, _BR, G), lambda i: (0, i, 0)),
          pl.BlockSpec((_BR, cp), lambda i: (i, 0)),
          pl.BlockSpec((1, cp), lambda i: (0, 0)),
      ],
      out_specs=pl.BlockSpec((_BR, cp), lambda i: (i, 0)),
      out_shape=jax.ShapeDtypeStruct((N, cp), jnp.float32),
  )


@jax.jit
def kernel(x, edge_index, pseudo, params):
  N = x.shape[0]
  E = edge_index.shape[1]
  src = edge_index[0]
  dst = edge_index[1]
  ps = pseudo[:, 0]
  zeros_ng = jnp.zeros((N, G), jnp.float32)

  deg_parts = _deg_kernel(N, E)(
      dst, zeros_ng, jnp.ones((C, G), jnp.float32))

  layer_dims = [(p["weight"].shape[1], p["weight"].shape[2], p["weight"].shape[0])
                for p in params]
  wcats = []
  for p, (cin, cout, K) in zip(params, layer_dims):
    wflat = jnp.transpose(p["weight"], (1, 0, 2)).reshape(cin, K * cout)
    wcats.append(jnp.concatenate([wflat, p["root"]], axis=1))

  cin0, cout0, k0 = layer_dims[0]
  z, rt = _mm0(N, cin0, k0, cout0)(x, wcats[0])

  for li, (cin, cout, K) in enumerate(layer_dims):
    parts = _edge_kernel(N, E, K, cout)(
        z.reshape(N * K, G), src, dst, ps, zeros_ng)
    bias = params[li]["bias"].reshape(1, cout)
    if li + 1 < len(layer_dims):
      cin_n, cout_n, k_n = layer_dims[li + 1]
      z, rt = _mm_fused(N, cout, k_n, cout_n, elu=li in (0, 2, 4))(
          parts, deg_parts, rt, bias, wcats[li + 1])
    else:
      return _final(N, cout)(parts, deg_parts, rt, bias)


# speedup vs baseline: 12.1064x; 1.1795x over previous
import functools

import jax
import jax.numpy as jnp
from jax import lax
from jax.experimental import pallas as pl
from jax.experimental.pallas import tpu as pltpu
from jax.experimental.pallas import tpu_sc as plsc

NC = 2
NS = 16
NW = NC * NS
C = 128
LANES = 16
G = 128


def _row_chunks(N):
  grans = N // 8
  gpt = (grans // NS) * 8
  gtail = grans - (grans // NS) * NS
  return gpt, gtail


def _edge_kernel(N, E, K, cout):
  paired = 2 * cout <= G
  CN = 96 if paired else 64
  per_tile = E // NW
  nfull = per_tile // CN
  tail = per_tile - nfull * CN
  assert nfull % 2 == 0 and tail % 8 == 0
  gpt, gtail = _row_chunks(N)
  jpr = cout // LANES
  mesh = plsc.VectorSubcoreMesh(core_axis_name="c", subcore_axis_name="s")

  scratch = [
      pltpu.VMEM((CN,), jnp.int32),
      pltpu.VMEM((2, CN), jnp.int32),
      pltpu.VMEM((CN,), jnp.float32),
      pltpu.VMEM((2, CN), jnp.int32),
      pltpu.VMEM((2, CN), jnp.int32),
      pltpu.VMEM((2, CN), jnp.float32),
      pltpu.VMEM((CN, G), jnp.float32),
      pltpu.VMEM((CN, G), jnp.float32),
      pltpu.VMEM((CN, G), jnp.float32),
      pltpu.VMEM((CN, G), jnp.float32),
      pltpu.VMEM((CN, G), jnp.float32),
      pltpu.VMEM((tail,), jnp.int32),
      pltpu.VMEM((tail,), jnp.int32),
      pltpu.VMEM((tail,), jnp.int32),
      pltpu.VMEM_SHARED((N, G), jnp.float32),
      pltpu.SemaphoreType.DMA,
      pltpu.SemaphoreType.DMA,
      pltpu.SemaphoreType.DMA,
  ]
  if paired:
    scratch[8] = pltpu.VMEM((8,), jnp.float32)
    scratch[9] = pltpu.VMEM((8,), jnp.float32)

  @functools.partial(
      pl.kernel,
      mesh=mesh,
      out_type=jax.ShapeDtypeStruct((2, N, G), jnp.float32),
      scratch_types=scratch,
  )
  def body(zf_hbm, src_hbm, dst_hbm, ps_hbm, zeros_hbm, out_hbm,
           src_v, dst_v, ps_v, g0_v, g1_v, f_v, r0a, r0b, r1a, r1b, msg,
           dst_t, g0_t, g1_t, acc, lsem, gsema, gsemb):
    cid = lax.axis_index("c")
    sid = lax.axis_index("s")
    wid = cid * NS + sid
    ebase = wid * per_tile
    r0s = (r0a, r0b)
    r1s = (r1a, r1b)
    gsems = (gsema, gsemb)

    pltpu.sync_copy(zeros_hbm.at[pl.ds(sid * gpt, gpt)],
                    acc.at[pl.ds(sid * gpt, gpt)])
    if gtail:
      @pl.when(sid < gtail)
      def _():
        t0 = NS * gpt + sid * 8
        pltpu.sync_copy(zeros_hbm.at[pl.ds(t0, 8)], acc.at[pl.ds(t0, 8)])

    if jpr < G // LANES:
      def zero_msg(e, carry):
        for j in range(jpr, G // LANES):
          msg[e, pl.ds(j * LANES, LANES)] = jnp.zeros((LANES,), jnp.float32)
        return carry
      lax.fori_loop(0, CN, zero_msg, 0)
    plsc.subcore_barrier()

    def lin_start(u, s):
      base = ebase + u * CN
      pltpu.async_copy(src_hbm.at[pl.ds(base, CN)], src_v, lsem)
      pltpu.async_copy(dst_hbm.at[pl.ds(base, CN)], dst_v.at[s], lsem)
      pltpu.async_copy(ps_hbm.at[pl.ds(base, CN)], ps_v, lsem)

    def lin_wait(u, s):
      base = ebase + u * CN
      pltpu.make_async_copy(src_hbm.at[pl.ds(base, CN)], src_v, lsem).wait()
      pltpu.make_async_copy(dst_hbm.at[pl.ds(base, CN)], dst_v.at[s],
                            lsem).wait()
      pltpu.make_async_copy(ps_hbm.at[pl.ds(base, CN)], ps_v, lsem).wait()

    def idx_compute(s):
      for j in range(CN // LANES):
        sl = pl.ds(j * LANES, LANES)
        v = ps_v[sl] * jnp.float32(K - 1)
        i0 = v.astype(jnp.int32)
        f_v[s, sl] = v - i0.astype(jnp.float32)
        row0 = src_v[sl] * K + i0
        g0_v[s, sl] = row0
        if not paired:
          g1_v[s, sl] = row0 + 1

    def gather_start(s):
      pltpu.async_copy(zf_hbm.at[g0_v.at[s]], r0s[s], gsems[s])
      if not paired:
        pltpu.async_copy(zf_hbm.at[g1_v.at[s]], r1s[s], gsems[s])

    def gather_wait(s):
      pltpu.make_async_copy(zf_hbm.at[g0_v.at[s]], r0s[s], gsems[s]).wait()
      if not paired:
        pltpu.make_async_copy(zf_hbm.at[g1_v.at[s]], r1s[s], gsems[s]).wait()

    def interp_scatter(s):
      r0 = r0s[s]
      r1 = r0 if paired else r1s[s]
      boff = cout if paired else 0

      def interp(t, carry):
        fg = f_v[s, pl.ds(t * LANES, LANES)]

        def inner(l4, carry2):
          for dl in range(4):
            l = l4 * 4 + dl
            fb = jnp.take_along_axis(
                fg, jnp.full((LANES,), l, jnp.int32), axis=0,
                mode="promise_in_bounds")
            e = t * LANES + l
            for j in range(jpr):
              a = r0[e, pl.ds(j * LANES, LANES)]
              b = r1[e, pl.ds(boff + j * LANES, LANES)]
              msg[e, pl.ds(j * LANES, LANES)] = a + fb * (b - a)
          return carry2

        lax.fori_loop(0, LANES // 4, inner, 0)
        return carry

      lax.fori_loop(0, CN // LANES, interp, 0)
      pltpu.sync_copy(msg, acc.at[dst_v.at[s]], add=True)

    pltpu.sync_copy(src_hbm.at[pl.ds(ebase, CN)], src_v)
    pltpu.sync_copy(dst_hbm.at[pl.ds(ebase, CN)], dst_v.at[0])
    pltpu.sync_copy(ps_hbm.at[pl.ds(ebase, CN)], ps_v)
    idx_compute(0)
    gather_start(0)
    if nfull > 1:
      lin_start(1, 1)

    def pipe_step(u, s):
      @pl.when(u + 1 < nfull)
      def _():
        lin_wait(u + 1, 1 - s)
        idx_compute(1 - s)
        gather_start(1 - s)
      gather_wait(s)
      interp_scatter(s)
      @pl.when(u + 2 < nfull)
      def _():
        lin_start(u + 2, s)

    def pipe_pair(v, carry):
      pipe_step(2 * v, 0)
      pipe_step(2 * v + 1, 1)
      return carry

    lax.fori_loop(0, nfull // 2, pipe_pair, 0)

    if tail:
      tb = ebase + nfull * CN
      pltpu.sync_copy(src_hbm.at[pl.ds(tb, tail)], src_v.at[pl.ds(0, tail)])
      pltpu.sync_copy(dst_hbm.at[pl.ds(tb, tail)], dst_t)
      pltpu.sync_copy(ps_hbm.at[pl.ds(tb, tail)], ps_v.at[pl.ds(0, tail)])
      for j in range(tail // LANES):
        sl = pl.ds(j * LANES, LANES)
        v = ps_v[sl] * jnp.float32(K - 1)
        i0 = v.astype(jnp.int32)
        f_v[0, sl] = v - i0.astype(jnp.float32)
        row0 = src_v[sl] * K + i0
        g0_t[sl] = row0
        if not paired:
          g1_t[sl] = row0 + 1
      pltpu.async_copy(zf_hbm.at[g0_t], r0a.at[pl.ds(0, tail)], gsema).wait()
      if not paired:
        pltpu.async_copy(zf_hbm.at[g1_t], r1a.at[pl.ds(0, tail)], gsemb).wait()
      boff = cout if paired else 0
      r1 = r0a if paired else r1a
      for t in range(tail // LANES):
        fg = f_v[0, pl.ds(t * LANES, LANES)]
        for l in range(LANES):
          fb = jnp.take_along_axis(
              fg, jnp.full((LANES,), l, jnp.int32), axis=0,
              mode="promise_in_bounds")
          e = t * LANES + l
          for j in range(jpr):
            a = r0a[e, pl.ds(j * LANES, LANES)]
            b = r1[e, pl.ds(boff + j * LANES, LANES)]
            msg[e, pl.ds(j * LANES, LANES)] = a + fb * (b - a)
      pltpu.sync_copy(msg.at[pl.ds(0, tail)], acc.at[dst_t], add=True)

    plsc.subcore_barrier()
    pltpu.sync_copy(acc.at[pl.ds(sid * gpt, gpt)],
                    out_hbm.at[cid, pl.ds(sid * gpt, gpt)])
    if gtail:
      @pl.when(sid < gtail)
      def _():
        t0 = NS * gpt + sid * 8
        pltpu.sync_copy(acc.at[pl.ds(t0, 8)], out_hbm.at[cid, pl.ds(t0, 8)])

  return body


def _deg_kernel(N, E):
  chunks = E // C
  nfull = chunks // NW
  rem = chunks - nfull * NW
  gpt, gtail = _row_chunks(N)
  mesh = plsc.VectorSubcoreMesh(core_axis_name="c", subcore_axis_name="s")

  @functools.partial(
      pl.kernel,
      mesh=mesh,
      out_type=jax.ShapeDtypeStruct((2, N, G), jnp.float32),
      scratch_types=[
          pltpu.VMEM((C,), jnp.int32),
          pltpu.VMEM((C, G), jnp.float32),
          pltpu.VMEM_SHARED((N, G), jnp.float32),
      ],
  )
  def body(dst_hbm, zeros_hbm, ones_hbm, out_hbm, dst_v, ones_v, acc):
    cid = lax.axis_index("c")
    sid = lax.axis_index("s")
    wid = cid * NS + sid

    pltpu.sync_copy(ones_hbm, ones_v)
    pltpu.sync_copy(zeros_hbm.at[pl.ds(sid * gpt, gpt)],
                    acc.at[pl.ds(sid * gpt, gpt)])
    if gtail:
      @pl.when(sid < gtail)
      def _():
        t0 = NS * gpt + sid * 8
        pltpu.sync_copy(zeros_hbm.at[pl.ds(t0, 8)], acc.at[pl.ds(t0, 8)])
    plsc.subcore_barrier()

    def do_chunk(c_idx):
      pltpu.sync_copy(dst_hbm.at[pl.ds(c_idx * C, C)], dst_v)
      pltpu.sync_copy(ones_v, acc.at[dst_v], add=True)

    def chunk_loop(t, carry):
      do_chunk(wid + NW * t)
      return carry

    lax.fori_loop(0, nfull, chunk_loop, 0)
    if rem:
      @pl.when(wid < rem)
      def _():
        do_chunk(NW * nfull + wid)

    plsc.subcore_barrier()
    pltpu.sync_copy(acc.at[pl.ds(sid * gpt, gpt)],
                    out_hbm.at[cid, pl.ds(sid * gpt, gpt)])
    if gtail:
      @pl.when(sid < gtail)
      def _():
        t0 = NS * gpt + sid * 8
        pltpu.sync_copy(acc.at[pl.ds(t0, 8)], out_hbm.at[cid, pl.ds(t0, 8)])

  return body


_BR = 1000


def _pad_table(z, K, cout):
  br = z.shape[0]
  if cout == G:
    return z
  pieces = []
  if 2 * cout <= G:
    zpad = jnp.zeros((br, G - 2 * cout), jnp.float32)
    zslot = jnp.zeros((br, cout), jnp.float32)
    for k in range(K):
      pieces.append(z[:, k * cout:(k + 1) * cout])
      pieces.append(z[:, (k + 1) * cout:(k + 2) * cout] if k + 1 < K
                    else zslot)
      if G > 2 * cout:
        pieces.append(zpad)
  else:
    zpad = jnp.zeros((br, G - cout), jnp.float32)
    for k in range(K):
      pieces.append(z[:, k * cout:(k + 1) * cout])
      pieces.append(zpad)
  return jnp.concatenate(pieces, axis=1)


def _mm0_body(x_ref, w_ref, o1_ref, o2_ref, *, K, cout):
  kc = K * cout
  z = jnp.dot(x_ref[...], w_ref[...], preferred_element_type=jnp.float32)
  o1_ref[...] = _pad_table(z[:, :kc], K, cout)
  o2_ref[...] = z[:, kc:]


def _epilogue(p_ref, dp_ref, rt_ref, b_ref, cp, elu):
  d = dp_ref[0, :, 0:1] + dp_ref[1, :, 0:1]
  inv = 1.0 / jnp.maximum(d, 1.0)
  h = (p_ref[0, :, :cp] + p_ref[1, :, :cp]) * inv + rt_ref[...] + b_ref[...]
  if elu:
    h = jnp.where(h > 0, h, jnp.exp(jnp.minimum(h, 0.0)) - 1.0)
  return h


def _mm_fused_body(p_ref, dp_ref, rt_ref, b_ref, w_ref, o1_ref, o2_ref,
                   *, cp, K, cout, elu):
  kc = K * cout
  h = _epilogue(p_ref, dp_ref, rt_ref, b_ref, cp, elu)
  z = jnp.dot(h, w_ref[...], preferred_element_type=jnp.float32)
  o1_ref[...] = _pad_table(z[:, :kc], K, cout)
  o2_ref[...] = z[:, kc:]


def _final_body(p_ref, dp_ref, rt_ref, b_ref, o_ref, *, cp):
  h = _epilogue(p_ref, dp_ref, rt_ref, b_ref, cp, False)
  m = jnp.max(h, axis=1, keepdims=True)
  l = h - m
  s = jnp.sum(jnp.exp(l), axis=1, keepdims=True)
  o_ref[...] = l - jnp.log(s)


def _mm0(N, cin, K, cout):
  cols = (K + 1) * cout
  return pl.pallas_call(
      functools.partial(_mm0_body, K=K, cout=cout),
      grid=(N // _BR,),
      in_specs=[
          pl.BlockSpec((_BR, cin), lambda i: (i, 0)),
          pl.BlockSpec((cin, cols), lambda i: (0, 0)),
      ],
      out_specs=[
          pl.BlockSpec((_BR, K * G), lambda i: (i, 0)),
          pl.BlockSpec((_BR, cout), lambda i: (i, 0)),
      ],
      out_shape=[
          jax.ShapeDtypeStruct((N, K * G), jnp.float32),
          jax.ShapeDtypeStruct((N, cout), jnp.float32),
      ],
  )


def _mm_fused(N, cp, K, cout, elu):
  cols = (K + 1) * cout
  return pl.pallas_call(
      functools.partial(_mm_fused_body, cp=cp, K=K, cout=cout, elu=elu),
      grid=(N // _BR,),
      in_specs=[
          pl.BlockSpec((2, _BR, G), lambda i: (0, i, 0)),
          pl.BlockSpec((2, _BR, G), lambda i: (0, i, 0)),
          pl.BlockSpec((_BR, cp), lambda i: (i, 0)),
          pl.BlockSpec((1, cp), lambda i: (0, 0)),
          pl.BlockSpec((cp, cols), lambda i: (0, 0)),
      ],
      out_specs=[
          pl.BlockSpec((_BR, K * G), lambda i: (i, 0)),
          pl.BlockSpec((_BR, cout), lambda i: (i, 0)),
      ],
      out_shape=[
          jax.ShapeDtypeStruct((N, K * G), jnp.float32),
          jax.ShapeDtypeStruct((N, cout), jnp.float32),
      ],
  )


def _final(N, cp):
  return pl.pallas_call(
      functools.partial(_final_body, cp=cp),
      grid=(N // _BR,),
      in_specs=[
          pl.BlockSpec((2, _BR, G), lambda i: (0, i, 0)),
          pl.BlockSpec((2, _BR, G), lambda i: (0, i, 0)),
          pl.BlockSpec((_BR, cp), lambda i: (i, 0)),
          pl.BlockSpec((1, cp), lambda i: (0, 0)),
      ],
      out_specs=pl.BlockSpec((_BR, cp), lambda i: (i, 0)),
      out_shape=jax.ShapeDtypeStruct((N, cp), jnp.float32),
  )


@jax.jit
def kernel(x, edge_index, pseudo, params):
  N = x.shape[0]
  E = edge_index.shape[1]
  src = edge_index[0]
  dst = edge_index[1]
  ps = pseudo[:, 0]
  zeros_ng = jnp.zeros((N, G), jnp.float32)

  deg_parts = _deg_kernel(N, E)(
      dst, zeros_ng, jnp.ones((C, G), jnp.float32))

  layer_dims = [(p["weight"].shape[1], p["weight"].shape[2], p["weight"].shape[0])
                for p in params]
  wcats = []
  for p, (cin, cout, K) in zip(params, layer_dims):
    wflat = jnp.transpose(p["weight"], (1, 0, 2)).reshape(cin, K * cout)
    wcats.append(jnp.concatenate([wflat, p["root"]], axis=1))

  cin0, cout0, k0 = layer_dims[0]
  z, rt = _mm0(N, cin0, k0, cout0)(x, wcats[0])

  for li, (cin, cout, K) in enumerate(layer_dims):
    parts = _edge_kernel(N, E, K, cout)(
        z.reshape(N * K, G), src, dst, ps, zeros_ng)
    bias = params[li]["bias"].reshape(1, cout)
    if li + 1 < len(layer_dims):
      cin_n, cout_n, k_n = layer_dims[li + 1]
      z, rt = _mm_fused(N, cout, k_n, cout_n, elu=li in (0, 2, 4))(
          parts, deg_parts, rt, bias, wcats[li + 1])
    else:
      return _final(N, cout)(parts, deg_parts, rt, bias)


# speedup vs baseline: 21.4109x; 1.7686x over previous
import functools

import jax
import jax.numpy as jnp
from jax import lax
from jax.experimental import pallas as pl
from jax.experimental.pallas import tpu as pltpu
from jax.experimental.pallas import tpu_sc as plsc

NC = 2
NS = 16
NW = NC * NS
C = 128
LANES = 16
G = 128


def _row_chunks(N):
  grans = N // 8
  gpt = (grans // NS) * 8
  gtail = grans - (grans // NS) * NS
  return gpt, gtail


def _edge_kernel(N, E, K, cout):
  paired = 2 * cout <= G
  CN = 96 if paired else 64
  per_tile = E // NW
  nfull = per_tile // CN
  tail = per_tile - nfull * CN
  assert nfull % 2 == 0 and tail % 8 == 0
  gpt, gtail = _row_chunks(N)
  jpr = cout // LANES
  mesh = plsc.VectorSubcoreMesh(core_axis_name="c", subcore_axis_name="s")

  scratch = [
      pltpu.VMEM((CN,), jnp.int32),
      pltpu.VMEM((2, CN), jnp.int32),
      pltpu.VMEM((CN,), jnp.float32),
      pltpu.VMEM((2, CN), jnp.int32),
      pltpu.VMEM((2, CN), jnp.int32),
      pltpu.VMEM((2, CN), jnp.float32),
      pltpu.VMEM((CN, G), jnp.float32),
      pltpu.VMEM((CN, G), jnp.float32),
      pltpu.VMEM((CN, G), jnp.float32),
      pltpu.VMEM((CN, G), jnp.float32),
      pltpu.VMEM((CN, G), jnp.float32),
      pltpu.VMEM((tail,), jnp.int32),
      pltpu.VMEM((tail,), jnp.int32),
      pltpu.VMEM((tail,), jnp.int32),
      pltpu.VMEM_SHARED((N, G), jnp.float32),
      pltpu.SemaphoreType.DMA,
      pltpu.SemaphoreType.DMA,
      pltpu.SemaphoreType.DMA,
  ]
  if paired:
    scratch[8] = pltpu.VMEM((8,), jnp.float32)
    scratch[9] = pltpu.VMEM((8,), jnp.float32)

  @functools.partial(
      pl.kernel,
      mesh=mesh,
      out_type=jax.ShapeDtypeStruct((2, N, G), jnp.float32),
      scratch_types=scratch,
  )
  def body(zf_hbm, src_hbm, dst_hbm, ps_hbm, zeros_hbm, out_hbm,
           src_v, dst_v, ps_v, g0_v, g1_v, f_v, r0a, r0b, r1a, r1b, msg,
           dst_t, g0_t, g1_t, acc, lsem, gsema, gsemb):
    cid = lax.axis_index("c")
    sid = lax.axis_index("s")
    wid = cid * NS + sid
    ebase = wid * per_tile
    r0s = (r0a, r0b)
    r1s = (r1a, r1b)
    gsems = (gsema, gsemb)

    pltpu.sync_copy(zeros_hbm.at[pl.ds(sid * gpt, gpt)],
                    acc.at[pl.ds(sid * gpt, gpt)])
    if gtail:
      @pl.when(sid < gtail)
      def _():
        t0 = NS * gpt + sid * 8
        pltpu.sync_copy(zeros_hbm.at[pl.ds(t0, 8)], acc.at[pl.ds(t0, 8)])

    if jpr < G // LANES:
      def zero_msg(e, carry):
        for j in range(jpr, G // LANES):
          msg[e, pl.ds(j * LANES, LANES)] = jnp.zeros((LANES,), jnp.float32)
        return carry
      lax.fori_loop(0, CN, zero_msg, 0)
    plsc.subcore_barrier()

    def lin_start(u, s):
      base = ebase + u * CN
      pltpu.async_copy(src_hbm.at[pl.ds(base, CN)], src_v, lsem)
      pltpu.async_copy(dst_hbm.at[pl.ds(base, CN)], dst_v.at[s], lsem)
      pltpu.async_copy(ps_hbm.at[pl.ds(base, CN)], ps_v, lsem)

    def lin_wait(u, s):
      base = ebase + u * CN
      pltpu.make_async_copy(src_hbm.at[pl.ds(base, CN)], src_v, lsem).wait()
      pltpu.make_async_copy(dst_hbm.at[pl.ds(base, CN)], dst_v.at[s],
                            lsem).wait()
      pltpu.make_async_copy(ps_hbm.at[pl.ds(base, CN)], ps_v, lsem).wait()

    def idx_compute(s):
      for j in range(CN // LANES):
        sl = pl.ds(j * LANES, LANES)
        v = ps_v[sl] * jnp.float32(K - 1)
        i0 = v.astype(jnp.int32)
        f_v[s, sl] = v - i0.astype(jnp.float32)
        row0 = src_v[sl] * K + i0
        g0_v[s, sl] = row0
        if not paired:
          g1_v[s, sl] = row0 + 1

    def gather_start(s):
      pltpu.async_copy(zf_hbm.at[g0_v.at[s]], r0s[s], gsems[s])
      if not paired:
        pltpu.async_copy(zf_hbm.at[g1_v.at[s]], r1s[s], gsems[s])

    def gather_wait(s):
      pltpu.make_async_copy(zf_hbm.at[g0_v.at[s]], r0s[s], gsems[s]).wait()
      if not paired:
        pltpu.make_async_copy(zf_hbm.at[g1_v.at[s]], r1s[s], gsems[s]).wait()

    def interp_scatter(s):
      r0 = r0s[s]
      r1 = r0 if paired else r1s[s]
      boff = cout if paired else 0

      @plsc.parallel_loop(0, CN // LANES, 1, unroll=2)
      def interp(t):
        fg = f_v[s, pl.ds(t * LANES, LANES)]

        @plsc.parallel_loop(0, LANES // 4, 1, unroll=2)
        def inner(l4):
          for dl in range(4):
            l = l4 * 4 + dl
            fb = jnp.take_along_axis(
                fg, jnp.full((LANES,), l, jnp.int32), axis=0,
                mode="promise_in_bounds")
            e = t * LANES + l
            for j in range(jpr):
              a = r0[e, pl.ds(j * LANES, LANES)]
              b = r1[e, pl.ds(boff + j * LANES, LANES)]
              msg[e, pl.ds(j * LANES, LANES)] = a + fb * (b - a)

      pltpu.sync_copy(msg, acc.at[dst_v.at[s]], add=True)

    pltpu.sync_copy(src_hbm.at[pl.ds(ebase, CN)], src_v)
    pltpu.sync_copy(dst_hbm.at[pl.ds(ebase, CN)], dst_v.at[0])
    pltpu.sync_copy(ps_hbm.at[pl.ds(ebase, CN)], ps_v)
    idx_compute(0)
    gather_start(0)
    if nfull > 1:
      lin_start(1, 1)

    def pipe_step(u, s):
      @pl.when(u + 1 < nfull)
      def _():
        lin_wait(u + 1, 1 - s)
        idx_compute(1 - s)
        gather_start(1 - s)
      gather_wait(s)
      interp_scatter(s)
      @pl.when(u + 2 < nfull)
      def _():
        lin_start(u + 2, s)

    def pipe_pair(v, carry):
      pipe_step(2 * v, 0)
      pipe_step(2 * v + 1, 1)
      return carry

    lax.fori_loop(0, nfull // 2, pipe_pair, 0)

    if tail:
      tb = ebase + nfull * CN
      pltpu.sync_copy(src_hbm.at[pl.ds(tb, tail)], src_v.at[pl.ds(0, tail)])
      pltpu.sync_copy(dst_hbm.at[pl.ds(tb, tail)], dst_t)
      pltpu.sync_copy(ps_hbm.at[pl.ds(tb, tail)], ps_v.at[pl.ds(0, tail)])
      for j in range(tail // LANES):
        sl = pl.ds(j * LANES, LANES)
        v = ps_v[sl] * jnp.float32(K - 1)
        i0 = v.astype(jnp.int32)
        f_v[0, sl] = v - i0.astype(jnp.float32)
        row0 = src_v[sl] * K + i0
        g0_t[sl] = row0
        if not paired:
          g1_t[sl] = row0 + 1
      pltpu.async_copy(zf_hbm.at[g0_t], r0a.at[pl.ds(0, tail)], gsema).wait()
      if not paired:
        pltpu.async_copy(zf_hbm.at[g1_t], r1a.at[pl.ds(0, tail)], gsemb).wait()
      boff = cout if paired else 0
      r1 = r0a if paired else r1a
      for t in range(tail // LANES):
        fg = f_v[0, pl.ds(t * LANES, LANES)]
        for l in range(LANES):
          fb = jnp.take_along_axis(
              fg, jnp.full((LANES,), l, jnp.int32), axis=0,
              mode="promise_in_bounds")
          e = t * LANES + l
          for j in range(jpr):
            a = r0a[e, pl.ds(j * LANES, LANES)]
            b = r1[e, pl.ds(boff + j * LANES, LANES)]
            msg[e, pl.ds(j * LANES, LANES)] = a + fb * (b - a)
      pltpu.sync_copy(msg.at[pl.ds(0, tail)], acc.at[dst_t], add=True)

    plsc.subcore_barrier()
    pltpu.sync_copy(acc.at[pl.ds(sid * gpt, gpt)],
                    out_hbm.at[cid, pl.ds(sid * gpt, gpt)])
    if gtail:
      @pl.when(sid < gtail)
      def _():
        t0 = NS * gpt + sid * 8
        pltpu.sync_copy(acc.at[pl.ds(t0, 8)], out_hbm.at[cid, pl.ds(t0, 8)])

  return body


def _deg_kernel(N, E):
  chunks = E // C
  nfull = chunks // NW
  rem = chunks - nfull * NW
  gpt, gtail = _row_chunks(N)
  mesh = plsc.VectorSubcoreMesh(core_axis_name="c", subcore_axis_name="s")

  @functools.partial(
      pl.kernel,
      mesh=mesh,
      out_type=jax.ShapeDtypeStruct((2, N, G), jnp.float32),
      scratch_types=[
          pltpu.VMEM((C,), jnp.int32),
          pltpu.VMEM((C, G), jnp.float32),
          pltpu.VMEM_SHARED((N, G), jnp.float32),
      ],
  )
  def body(dst_hbm, zeros_hbm, ones_hbm, out_hbm, dst_v, ones_v, acc):
    cid = lax.axis_index("c")
    sid = lax.axis_index("s")
    wid = cid * NS + sid

    pltpu.sync_copy(ones_hbm, ones_v)
    pltpu.sync_copy(zeros_hbm.at[pl.ds(sid * gpt, gpt)],
                    acc.at[pl.ds(sid * gpt, gpt)])
    if gtail:
      @pl.when(sid < gtail)
      def _():
        t0 = NS * gpt + sid * 8
        pltpu.sync_copy(zeros_hbm.at[pl.ds(t0, 8)], acc.at[pl.ds(t0, 8)])
    plsc.subcore_barrier()

    def do_chunk(c_idx):
      pltpu.sync_copy(dst_hbm.at[pl.ds(c_idx * C, C)], dst_v)
      pltpu.sync_copy(ones_v, acc.at[dst_v], add=True)

    def chunk_loop(t, carry):
      do_chunk(wid + NW * t)
      return carry

    lax.fori_loop(0, nfull, chunk_loop, 0)
    if rem:
      @pl.when(wid < rem)
      def _():
        do_chunk(NW * nfull + wid)

    plsc.subcore_barrier()
    pltpu.sync_copy(acc.at[pl.ds(sid * gpt, gpt)],
                    out_hbm.at[cid, pl.ds(sid * gpt, gpt)])
    if gtail:
      @pl.when(sid < gtail)
      def _():
        t0 = NS * gpt + sid * 8
        pltpu.sync_copy(acc.at[pl.ds(t0, 8)], out_hbm.at[cid, pl.ds(t0, 8)])

  return body


_BR = 1000


def _pad_table(z, K, cout):
  br = z.shape[0]
  if cout == G:
    return z
  pieces = []
  if 2 * cout <= G:
    zpad = jnp.zeros((br, G - 2 * cout), jnp.float32)
    zslot = jnp.zeros((br, cout), jnp.float32)
    for k in range(K):
      pieces.append(z[:, k * cout:(k + 1) * cout])
      pieces.append(z[:, (k + 1) * cout:(k + 2) * cout] if k + 1 < K
                    else zslot)
      if G > 2 * cout:
        pieces.append(zpad)
  else:
    zpad = jnp.zeros((br, G - cout), jnp.float32)
    for k in range(K):
      pieces.append(z[:, k * cout:(k + 1) * cout])
      pieces.append(zpad)
  return jnp.concatenate(pieces, axis=1)


def _mm0_body(x_ref, w_ref, o1_ref, o2_ref, *, K, cout):
  kc = K * cout
  z = jnp.dot(x_ref[...], w_ref[...], preferred_element_type=jnp.float32)
  o1_ref[...] = _pad_table(z[:, :kc], K, cout)
  o2_ref[...] = z[:, kc:]


def _epilogue(p_ref, dp_ref, rt_ref, b_ref, cp, elu):
  d = dp_ref[0, :, 0:1] + dp_ref[1, :, 0:1]
  inv = 1.0 / jnp.maximum(d, 1.0)
  h = (p_ref[0, :, :cp] + p_ref[1, :, :cp]) * inv + rt_ref[...] + b_ref[...]
  if elu:
    h = jnp.where(h > 0, h, jnp.exp(jnp.minimum(h, 0.0)) - 1.0)
  return h


def _mm_fused_body(p_ref, dp_ref, rt_ref, b_ref, w_ref, o1_ref, o2_ref,
                   *, cp, K, cout, elu):
  kc = K * cout
  h = _epilogue(p_ref, dp_ref, rt_ref, b_ref, cp, elu)
  z = jnp.dot(h, w_ref[...], preferred_element_type=jnp.float32)
  o1_ref[...] = _pad_table(z[:, :kc], K, cout)
  o2_ref[...] = z[:, kc:]


def _final_body(p_ref, dp_ref, rt_ref, b_ref, o_ref, *, cp):
  h = _epilogue(p_ref, dp_ref, rt_ref, b_ref, cp, False)
  m = jnp.max(h, axis=1, keepdims=True)
  l = h - m
  s = jnp.sum(jnp.exp(l), axis=1, keepdims=True)
  o_ref[...] = l - jnp.log(s)


def _mm0(N, cin, K, cout):
  cols = (K + 1) * cout
  return pl.pallas_call(
      functools.partial(_mm0_body, K=K, cout=cout),
      grid=(N // _BR,),
      in_specs=[
          pl.BlockSpec((_BR, cin), lambda i: (i, 0)),
          pl.BlockSpec((cin, cols), lambda i: (0, 0)),
      ],
      out_specs=[
          pl.BlockSpec((_BR, K * G), lambda i: (i, 0)),
          pl.BlockSpec((_BR, cout), lambda i: (i, 0)),
      ],
      out_shape=[
          jax.ShapeDtypeStruct((N, K * G), jnp.float32),
          jax.ShapeDtypeStruct((N, cout), jnp.float32),
      ],
  )


def _mm_fused(N, cp, K, cout, elu):
  cols = (K + 1) * cout
  return pl.pallas_call(
      functools.partial(_mm_fused_body, cp=cp, K=K, cout=cout, elu=elu),
      grid=(N // _BR,),
      in_specs=[
          pl.BlockSpec((2, _BR, G), lambda i: (0, i, 0)),
          pl.BlockSpec((2, _BR, G), lambda i: (0, i, 0)),
          pl.BlockSpec((_BR, cp), lambda i: (i, 0)),
          pl.BlockSpec((1, cp), lambda i: (0, 0)),
          pl.BlockSpec((cp, cols), lambda i: (0, 0)),
      ],
      out_specs=[
          pl.BlockSpec((_BR, K * G), lambda i: (i, 0)),
          pl.BlockSpec((_BR, cout), lambda i: (i, 0)),
      ],
      out_shape=[
          jax.ShapeDtypeStruct((N, K * G), jnp.float32),
          jax.ShapeDtypeStruct((N, cout), jnp.float32),
      ],
  )


def _final(N, cp):
  return pl.pallas_call(
      functools.partial(_final_body, cp=cp),
      grid=(N // _BR,),
      in_specs=[
          pl.BlockSpec((2, _BR, G), lambda i: (0, i, 0)),
          pl.BlockSpec((2, _BR, G), lambda i: (0, i, 0)),
          pl.BlockSpec((_BR, cp), lambda i: (i, 0)),
          pl.BlockSpec((1, cp), lambda i: (0, 0)),
      ],
      out_specs=pl.BlockSpec((_BR, cp), lambda i: (i, 0)),
      out_shape=jax.ShapeDtypeStruct((N, cp), jnp.float32),
  )


@jax.jit
def kernel(x, edge_index, pseudo, params):
  N = x.shape[0]
  E = edge_index.shape[1]
  src = edge_index[0]
  dst = edge_index[1]
  ps = pseudo[:, 0]
  zeros_ng = jnp.zeros((N, G), jnp.float32)

  deg_parts = _deg_kernel(N, E)(
      dst, zeros_ng, jnp.ones((C, G), jnp.float32))

  layer_dims = [(p["weight"].shape[1], p["weight"].shape[2], p["weight"].shape[0])
                for p in params]
  wcats = []
  for p, (cin, cout, K) in zip(params, layer_dims):
    wflat = jnp.transpose(p["weight"], (1, 0, 2)).reshape(cin, K * cout)
    wcats.append(jnp.concatenate([wflat, p["root"]], axis=1))

  cin0, cout0, k0 = layer_dims[0]
  z, rt = _mm0(N, cin0, k0, cout0)(x, wcats[0])

  for li, (cin, cout, K) in enumerate(layer_dims):
    parts = _edge_kernel(N, E, K, cout)(
        z.reshape(N * K, G), src, dst, ps, zeros_ng)
    bias = params[li]["bias"].reshape(1, cout)
    if li + 1 < len(layer_dims):
      cin_n, cout_n, k_n = layer_dims[li + 1]
      z, rt = _mm_fused(N, cout, k_n, cout_n, elu=li in (0, 2, 4))(
          parts, deg_parts, rt, bias, wcats[li + 1])
    else:
      return _final(N, cout)(parts, deg_parts, rt, bias)


# speedup vs baseline: 22.7396x; 1.0621x over previous
import functools

import jax
import jax.numpy as jnp
from jax import lax
from jax.experimental import pallas as pl
from jax.experimental.pallas import tpu as pltpu
from jax.experimental.pallas import tpu_sc as plsc

NC = 2
NS = 16
NW = NC * NS
C = 128
LANES = 16
G = 128


def _row_chunks(N):
  grans = N // 8
  gpt = (grans // NS) * 8
  gtail = grans - (grans // NS) * NS
  return gpt, gtail


def _edge_kernel(N, E, K, cout, with_deg=False):
  paired = 2 * cout <= G
  CN = 96 if paired else 64
  per_tile = E // NW
  nfull = per_tile // CN
  tail = per_tile - nfull * CN
  assert nfull % 2 == 0 and tail % 8 == 0
  gpt, gtail = _row_chunks(N)
  jpr = cout // LANES
  mesh = plsc.VectorSubcoreMesh(core_axis_name="c", subcore_axis_name="s")

  scratch = [
      pltpu.VMEM((CN,), jnp.int32),
      pltpu.VMEM((2, CN), jnp.int32),
      pltpu.VMEM((CN,), jnp.float32),
      pltpu.VMEM((2, CN), jnp.int32),
      pltpu.VMEM((2, CN), jnp.int32),
      pltpu.VMEM((2, CN), jnp.float32),
      pltpu.VMEM((CN, G), jnp.float32),
      pltpu.VMEM((CN, G), jnp.float32),
      pltpu.VMEM((CN, G), jnp.float32),
      pltpu.VMEM((CN, G), jnp.float32),
      pltpu.VMEM((CN, G), jnp.float32),
      pltpu.VMEM((tail,), jnp.int32),
      pltpu.VMEM((tail,), jnp.int32),
      pltpu.VMEM((tail,), jnp.int32),
      pltpu.VMEM_SHARED((N, G), jnp.float32),
      pltpu.SemaphoreType.DMA,
      pltpu.SemaphoreType.DMA,
      pltpu.SemaphoreType.DMA,
  ]
  if paired:
    scratch[8] = pltpu.VMEM((8,), jnp.float32)
    scratch[9] = pltpu.VMEM((8,), jnp.float32)

  @functools.partial(
      pl.kernel,
      mesh=mesh,
      out_type=jax.ShapeDtypeStruct((2, N, G), jnp.float32),
      scratch_types=scratch,
  )
  def body(zf_hbm, src_hbm, dst_hbm, ps_hbm, zeros_hbm, out_hbm,
           src_v, dst_v, ps_v, g0_v, g1_v, f_v, r0a, r0b, r1a, r1b, msg,
           dst_t, g0_t, g1_t, acc, lsem, gsema, gsemb):
    cid = lax.axis_index("c")
    sid = lax.axis_index("s")
    wid = cid * NS + sid
    ebase = wid * per_tile
    r0s = (r0a, r0b)
    r1s = (r1a, r1b)
    gsems = (gsema, gsemb)

    pltpu.sync_copy(zeros_hbm.at[pl.ds(sid * gpt, gpt)],
                    acc.at[pl.ds(sid * gpt, gpt)])
    if gtail:
      @pl.when(sid < gtail)
      def _():
        t0 = NS * gpt + sid * 8
        pltpu.sync_copy(zeros_hbm.at[pl.ds(t0, 8)], acc.at[pl.ds(t0, 8)])

    if jpr < G // LANES:
      deg_row = jnp.where(lax.iota(jnp.int32, LANES) == 0,
                          jnp.float32(1.0 if with_deg else 0.0),
                          jnp.float32(0.0))
      def zero_msg(e, carry):
        msg[e, pl.ds(jpr * LANES, LANES)] = deg_row
        for j in range(jpr + 1, G // LANES):
          msg[e, pl.ds(j * LANES, LANES)] = jnp.zeros((LANES,), jnp.float32)
        return carry
      lax.fori_loop(0, CN, zero_msg, 0)
    plsc.subcore_barrier()

    def lin_start(u, s):
      base = ebase + u * CN
      pltpu.async_copy(src_hbm.at[pl.ds(base, CN)], src_v, lsem)
      pltpu.async_copy(dst_hbm.at[pl.ds(base, CN)], dst_v.at[s], lsem)
      pltpu.async_copy(ps_hbm.at[pl.ds(base, CN)], ps_v, lsem)

    def lin_wait(u, s):
      base = ebase + u * CN
      pltpu.make_async_copy(src_hbm.at[pl.ds(base, CN)], src_v, lsem).wait()
      pltpu.make_async_copy(dst_hbm.at[pl.ds(base, CN)], dst_v.at[s],
                            lsem).wait()
      pltpu.make_async_copy(ps_hbm.at[pl.ds(base, CN)], ps_v, lsem).wait()

    def idx_compute(s):
      for j in range(CN // LANES):
        sl = pl.ds(j * LANES, LANES)
        v = ps_v[sl] * jnp.float32(K - 1)
        i0 = v.astype(jnp.int32)
        f_v[s, sl] = v - i0.astype(jnp.float32)
        row0 = src_v[sl] * K + i0
        g0_v[s, sl] = row0
        if not paired:
          g1_v[s, sl] = row0 + 1

    def gather_start(s):
      pltpu.async_copy(zf_hbm.at[g0_v.at[s]], r0s[s], gsems[s])
      if not paired:
        pltpu.async_copy(zf_hbm.at[g1_v.at[s]], r1s[s], gsems[s])

    def gather_wait(s):
      pltpu.make_async_copy(zf_hbm.at[g0_v.at[s]], r0s[s], gsems[s]).wait()
      if not paired:
        pltpu.make_async_copy(zf_hbm.at[g1_v.at[s]], r1s[s], gsems[s]).wait()

    def interp_scatter(s):
      r0 = r0s[s]
      r1 = r0 if paired else r1s[s]
      boff = cout if paired else 0

      @plsc.parallel_loop(0, CN // LANES, 1, unroll=2)
      def interp(t):
        fg = f_v[s, pl.ds(t * LANES, LANES)]

        @plsc.parallel_loop(0, LANES // 4, 1, unroll=2)
        def inner(l4):
          for dl in range(4):
            l = l4 * 4 + dl
            fb = jnp.take_along_axis(
                fg, jnp.full((LANES,), l, jnp.int32), axis=0,
                mode="promise_in_bounds")
            e = t * LANES + l
            for j in range(jpr):
              a = r0[e, pl.ds(j * LANES, LANES)]
              b = r1[e, pl.ds(boff + j * LANES, LANES)]
              msg[e, pl.ds(j * LANES, LANES)] = a + fb * (b - a)

      pltpu.sync_copy(msg, acc.at[dst_v.at[s]], add=True)

    pltpu.sync_copy(src_hbm.at[pl.ds(ebase, CN)], src_v)
    pltpu.sync_copy(dst_hbm.at[pl.ds(ebase, CN)], dst_v.at[0])
    pltpu.sync_copy(ps_hbm.at[pl.ds(ebase, CN)], ps_v)
    idx_compute(0)
    gather_start(0)
    if nfull > 1:
      lin_start(1, 1)

    def pipe_step(u, s):
      @pl.when(u + 1 < nfull)
      def _():
        lin_wait(u + 1, 1 - s)
        idx_compute(1 - s)
        gather_start(1 - s)
      gather_wait(s)
      interp_scatter(s)
      @pl.when(u + 2 < nfull)
      def _():
        lin_start(u + 2, s)

    def pipe_pair(v, carry):
      pipe_step(2 * v, 0)
      pipe_step(2 * v + 1, 1)
      return carry

    lax.fori_loop(0, nfull // 2, pipe_pair, 0)

    if tail:
      tb = ebase + nfull * CN
      pltpu.sync_copy(src_hbm.at[pl.ds(tb, tail)], src_v.at[pl.ds(0, tail)])
      pltpu.sync_copy(dst_hbm.at[pl.ds(tb, tail)], dst_t)
      pltpu.sync_copy(ps_hbm.at[pl.ds(tb, tail)], ps_v.at[pl.ds(0, tail)])
      for j in range(tail // LANES):
        sl = pl.ds(j * LANES, LANES)
        v = ps_v[sl] * jnp.float32(K - 1)
        i0 = v.astype(jnp.int32)
        f_v[0, sl] = v - i0.astype(jnp.float32)
        row0 = src_v[sl] * K + i0
        g0_t[sl] = row0
        if not paired:
          g1_t[sl] = row0 + 1
      pltpu.async_copy(zf_hbm.at[g0_t], r0a.at[pl.ds(0, tail)], gsema).wait()
      if not paired:
        pltpu.async_copy(zf_hbm.at[g1_t], r1a.at[pl.ds(0, tail)], gsemb).wait()
      boff = cout if paired else 0
      r1 = r0a if paired else r1a
      for t in range(tail // LANES):
        fg = f_v[0, pl.ds(t * LANES, LANES)]
        for l in range(LANES):
          fb = jnp.take_along_axis(
              fg, jnp.full((LANES,), l, jnp.int32), axis=0,
              mode="promise_in_bounds")
          e = t * LANES + l
          for j in range(jpr):
            a = r0a[e, pl.ds(j * LANES, LANES)]
            b = r1[e, pl.ds(boff + j * LANES, LANES)]
            msg[e, pl.ds(j * LANES, LANES)] = a + fb * (b - a)
      pltpu.sync_copy(msg.at[pl.ds(0, tail)], acc.at[dst_t], add=True)

    plsc.subcore_barrier()
    pltpu.sync_copy(acc.at[pl.ds(sid * gpt, gpt)],
                    out_hbm.at[cid, pl.ds(sid * gpt, gpt)])
    if gtail:
      @pl.when(sid < gtail)
      def _():
        t0 = NS * gpt + sid * 8
        pltpu.sync_copy(acc.at[pl.ds(t0, 8)], out_hbm.at[cid, pl.ds(t0, 8)])

  return body


def _deg_kernel(N, E):
  chunks = E // C
  nfull = chunks // NW
  rem = chunks - nfull * NW
  gpt, gtail = _row_chunks(N)
  mesh = plsc.VectorSubcoreMesh(core_axis_name="c", subcore_axis_name="s")

  @functools.partial(
      pl.kernel,
      mesh=mesh,
      out_type=jax.ShapeDtypeStruct((2, N, G), jnp.float32),
      scratch_types=[
          pltpu.VMEM((C,), jnp.int32),
          pltpu.VMEM((C, G), jnp.float32),
          pltpu.VMEM_SHARED((N, G), jnp.float32),
      ],
  )
  def body(dst_hbm, zeros_hbm, ones_hbm, out_hbm, dst_v, ones_v, acc):
    cid = lax.axis_index("c")
    sid = lax.axis_index("s")
    wid = cid * NS + sid

    pltpu.sync_copy(ones_hbm, ones_v)
    pltpu.sync_copy(zeros_hbm.at[pl.ds(sid * gpt, gpt)],
                    acc.at[pl.ds(sid * gpt, gpt)])
    if gtail:
      @pl.when(sid < gtail)
      def _():
        t0 = NS * gpt + sid * 8
        pltpu.sync_copy(zeros_hbm.at[pl.ds(t0, 8)], acc.at[pl.ds(t0, 8)])
    plsc.subcore_barrier()

    def do_chunk(c_idx):
      pltpu.sync_copy(dst_hbm.at[pl.ds(c_idx * C, C)], dst_v)
      pltpu.sync_copy(ones_v, acc.at[dst_v], add=True)

    def chunk_loop(t, carry):
      do_chunk(wid + NW * t)
      return carry

    lax.fori_loop(0, nfull, chunk_loop, 0)
    if rem:
      @pl.when(wid < rem)
      def _():
        do_chunk(NW * nfull + wid)

    plsc.subcore_barrier()
    pltpu.sync_copy(acc.at[pl.ds(sid * gpt, gpt)],
                    out_hbm.at[cid, pl.ds(sid * gpt, gpt)])
    if gtail:
      @pl.when(sid < gtail)
      def _():
        t0 = NS * gpt + sid * 8
        pltpu.sync_copy(acc.at[pl.ds(t0, 8)], out_hbm.at[cid, pl.ds(t0, 8)])

  return body


_BR = 1000


def _pad_table(z, K, cout):
  br = z.shape[0]
  if cout == G:
    return z
  pieces = []
  if 2 * cout <= G:
    zpad = jnp.zeros((br, G - 2 * cout), jnp.float32)
    zslot = jnp.zeros((br, cout), jnp.float32)
    for k in range(K):
      pieces.append(z[:, k * cout:(k + 1) * cout])
      pieces.append(z[:, (k + 1) * cout:(k + 2) * cout] if k + 1 < K
                    else zslot)
      if G > 2 * cout:
        pieces.append(zpad)
  else:
    zpad = jnp.zeros((br, G - cout), jnp.float32)
    for k in range(K):
      pieces.append(z[:, k * cout:(k + 1) * cout])
      pieces.append(zpad)
  return jnp.concatenate(pieces, axis=1)


def _mm0_body(x_ref, w_ref, o1_ref, o2_ref, *, K, cout):
  kc = K * cout
  z = jnp.dot(x_ref[...], w_ref[...], preferred_element_type=jnp.float32)
  o1_ref[...] = _pad_table(z[:, :kc], K, cout)
  o2_ref[...] = z[:, kc:]


def _epilogue(p_ref, dp_ref, rt_ref, b_ref, cp, elu, dlane):
  d = dp_ref[0, :, dlane:dlane + 1] + dp_ref[1, :, dlane:dlane + 1]
  inv = 1.0 / jnp.maximum(d, 1.0)
  h = (p_ref[0, :, :cp] + p_ref[1, :, :cp]) * inv + rt_ref[...] + b_ref[...]
  if elu:
    h = jnp.where(h > 0, h, jnp.exp(jnp.minimum(h, 0.0)) - 1.0)
  return h


def _mm_fused_body(p_ref, dp_ref, rt_ref, b_ref, w_ref, o1_ref, o2_ref,
                   *, cp, K, cout, elu, dlane):
  kc = K * cout
  h = _epilogue(p_ref, dp_ref, rt_ref, b_ref, cp, elu, dlane)
  z = jnp.dot(h, w_ref[...], preferred_element_type=jnp.float32)
  o1_ref[...] = _pad_table(z[:, :kc], K, cout)
  o2_ref[...] = z[:, kc:]


def _final_body(p_ref, dp_ref, rt_ref, b_ref, o_ref, *, cp, dlane):
  h = _epilogue(p_ref, dp_ref, rt_ref, b_ref, cp, False, dlane)
  m = jnp.max(h, axis=1, keepdims=True)
  l = h - m
  s = jnp.sum(jnp.exp(l), axis=1, keepdims=True)
  o_ref[...] = l - jnp.log(s)


def _mm0(N, cin, K, cout):
  cols = (K + 1) * cout
  return pl.pallas_call(
      functools.partial(_mm0_body, K=K, cout=cout),
      grid=(N // _BR,),
      in_specs=[
          pl.BlockSpec((_BR, cin), lambda i: (i, 0)),
          pl.BlockSpec((cin, cols), lambda i: (0, 0)),
      ],
      out_specs=[
          pl.BlockSpec((_BR, K * G), lambda i: (i, 0)),
          pl.BlockSpec((_BR, cout), lambda i: (i, 0)),
      ],
      out_shape=[
          jax.ShapeDtypeStruct((N, K * G), jnp.float32),
          jax.ShapeDtypeStruct((N, cout), jnp.float32),
      ],
  )


def _mm_fused(N, cp, K, cout, elu, dlane):
  cols = (K + 1) * cout
  return pl.pallas_call(
      functools.partial(_mm_fused_body, cp=cp, K=K, cout=cout, elu=elu,
                        dlane=dlane),
      grid=(N // _BR,),
      in_specs=[
          pl.BlockSpec((2, _BR, G), lambda i: (0, i, 0)),
          pl.BlockSpec((2, _BR, G), lambda i: (0, i, 0)),
          pl.BlockSpec((_BR, cp), lambda i: (i, 0)),
          pl.BlockSpec((1, cp), lambda i: (0, 0)),
          pl.BlockSpec((cp, cols), lambda i: (0, 0)),
      ],
      out_specs=[
          pl.BlockSpec((_BR, K * G), lambda i: (i, 0)),
          pl.BlockSpec((_BR, cout), lambda i: (i, 0)),
      ],
      out_shape=[
          jax.ShapeDtypeStruct((N, K * G), jnp.float32),
          jax.ShapeDtypeStruct((N, cout), jnp.float32),
      ],
  )


def _final(N, cp, dlane):
  return pl.pallas_call(
      functools.partial(_final_body, cp=cp, dlane=dlane),
      grid=(N // _BR,),
      in_specs=[
          pl.BlockSpec((2, _BR, G), lambda i: (0, i, 0)),
          pl.BlockSpec((2, _BR, G), lambda i: (0, i, 0)),
          pl.BlockSpec((_BR, cp), lambda i: (i, 0)),
          pl.BlockSpec((1, cp), lambda i: (0, 0)),
      ],
      out_specs=pl.BlockSpec((_BR, cp), lambda i: (i, 0)),
      out_shape=jax.ShapeDtypeStruct((N, cp), jnp.float32),
  )


@jax.jit
def kernel(x, edge_index, pseudo, params):
  N = x.shape[0]
  E = edge_index.shape[1]
  src = edge_index[0]
  dst = edge_index[1]
  ps = pseudo[:, 0]
  zeros_ng = jnp.zeros((N, G), jnp.float32)

  layer_dims = [(p["weight"].shape[1], p["weight"].shape[2], p["weight"].shape[0])
                for p in params]
  wcats = []
  for p, (cin, cout, K) in zip(params, layer_dims):
    wflat = jnp.transpose(p["weight"], (1, 0, 2)).reshape(cin, K * cout)
    wcats.append(jnp.concatenate([wflat, p["root"]], axis=1))

  cin0, cout0, k0 = layer_dims[0]
  z, rt = _mm0(N, cin0, k0, cout0)(x, wcats[0])

  deg_parts = None
  dlane = layer_dims[0][1]
  for li, (cin, cout, K) in enumerate(layer_dims):
    parts = _edge_kernel(N, E, K, cout, with_deg=(li == 0))(
        z.reshape(N * K, G), src, dst, ps, zeros_ng)
    if li == 0:
      deg_parts = parts
    bias = params[li]["bias"].reshape(1, cout)
    if li + 1 < len(layer_dims):
      cin_n, cout_n, k_n = layer_dims[li + 1]
      z, rt = _mm_fused(N, cout, k_n, cout_n, elu=li in (0, 2, 4),
                        dlane=dlane)(
          parts, deg_parts, rt, bias, wcats[li + 1])
    else:
      return _final(N, cout, dlane)(parts, deg_parts, rt, bias)


# speedup vs baseline: 23.8313x; 1.0480x over previous
import functools

import jax
import jax.numpy as jnp
from jax import lax
from jax.experimental import pallas as pl
from jax.experimental.pallas import tpu as pltpu
from jax.experimental.pallas import tpu_sc as plsc

NC = 2
NS = 16
NW = NC * NS
C = 128
LANES = 16
G = 128


def _row_chunks(N):
  grans = N // 8
  gpt = (grans // NS) * 8
  gtail = grans - (grans // NS) * NS
  return gpt, gtail


def _edge_kernel(N, E, K, cout, with_deg=False):
  paired = 2 * cout <= G
  W = 2 * cout if paired else G
  WA = cout + LANES if with_deg else (cout if paired else G)
  CN = 128 if paired else 64
  per_tile = E // NW
  nfull = per_tile // CN
  tail = per_tile - nfull * CN
  assert nfull % 2 == 0 and tail % 8 == 0
  gpt, gtail = _row_chunks(N)
  jpr = cout // LANES
  mesh = plsc.VectorSubcoreMesh(core_axis_name="c", subcore_axis_name="s")

  scratch = [
      pltpu.VMEM((CN,), jnp.int32),
      pltpu.VMEM((2, CN), jnp.int32),
      pltpu.VMEM((CN,), jnp.float32),
      pltpu.VMEM((2, CN), jnp.int32),
      pltpu.VMEM((2, CN), jnp.int32),
      pltpu.VMEM((2, CN), jnp.float32),
      pltpu.VMEM((CN, W), jnp.float32),
      pltpu.VMEM((CN, W), jnp.float32),
      pltpu.VMEM((CN, W), jnp.float32),
      pltpu.VMEM((CN, W), jnp.float32),
      pltpu.VMEM((CN, WA), jnp.float32),
      pltpu.VMEM((tail,), jnp.int32),
      pltpu.VMEM((tail,), jnp.int32),
      pltpu.VMEM((tail,), jnp.int32),
      pltpu.VMEM_SHARED((N, WA), jnp.float32),
      pltpu.SemaphoreType.DMA,
      pltpu.SemaphoreType.DMA,
      pltpu.SemaphoreType.DMA,
  ]
  if paired:
    scratch[8] = pltpu.VMEM((8,), jnp.float32)
    scratch[9] = pltpu.VMEM((8,), jnp.float32)

  @functools.partial(
      pl.kernel,
      mesh=mesh,
      out_type=jax.ShapeDtypeStruct((2, N, WA), jnp.float32),
      scratch_types=scratch,
      compiler_params=(pltpu.CompilerParams(use_tc_tiling_on_sc=False)
                       if paired else None),
  )
  def body(zf_hbm, src_hbm, dst_hbm, ps_hbm, zeros_hbm, out_hbm,
           src_v, dst_v, ps_v, g0_v, g1_v, f_v, r0a, r0b, r1a, r1b, msg,
           dst_t, g0_t, g1_t, acc, lsem, gsema, gsemb):
    cid = lax.axis_index("c")
    sid = lax.axis_index("s")
    wid = cid * NS + sid
    ebase = wid * per_tile
    r0s = (r0a, r0b)
    r1s = (r1a, r1b)
    gsems = (gsema, gsemb)

    pltpu.sync_copy(zeros_hbm.at[pl.ds(sid * gpt, gpt)],
                    acc.at[pl.ds(sid * gpt, gpt)])
    if gtail:
      @pl.when(sid < gtail)
      def _():
        t0 = NS * gpt + sid * 8
        pltpu.sync_copy(zeros_hbm.at[pl.ds(t0, 8)], acc.at[pl.ds(t0, 8)])

    if jpr < WA // LANES:
      deg_row = jnp.where(lax.iota(jnp.int32, LANES) == 0,
                          jnp.float32(1.0 if with_deg else 0.0),
                          jnp.float32(0.0))
      def zero_msg(e, carry):
        msg[e, pl.ds(jpr * LANES, LANES)] = deg_row
        for j in range(jpr + 1, WA // LANES):
          msg[e, pl.ds(j * LANES, LANES)] = jnp.zeros((LANES,), jnp.float32)
        return carry
      lax.fori_loop(0, CN, zero_msg, 0)
    plsc.subcore_barrier()

    def lin_start(u, s):
      base = ebase + u * CN
      pltpu.async_copy(src_hbm.at[pl.ds(base, CN)], src_v, lsem)
      pltpu.async_copy(dst_hbm.at[pl.ds(base, CN)], dst_v.at[s], lsem)
      pltpu.async_copy(ps_hbm.at[pl.ds(base, CN)], ps_v, lsem)

    def lin_wait(u, s):
      base = ebase + u * CN
      pltpu.make_async_copy(src_hbm.at[pl.ds(base, CN)], src_v, lsem).wait()
      pltpu.make_async_copy(dst_hbm.at[pl.ds(base, CN)], dst_v.at[s],
                            lsem).wait()
      pltpu.make_async_copy(ps_hbm.at[pl.ds(base, CN)], ps_v, lsem).wait()

    def idx_compute(s):
      for j in range(CN // LANES):
        sl = pl.ds(j * LANES, LANES)
        v = ps_v[sl] * jnp.float32(K - 1)
        i0 = v.astype(jnp.int32)
        f_v[s, sl] = v - i0.astype(jnp.float32)
        row0 = src_v[sl] * K + i0
        g0_v[s, sl] = row0
        if not paired:
          g1_v[s, sl] = row0 + 1

    def gather_start(s):
      pltpu.async_copy(zf_hbm.at[g0_v.at[s]], r0s[s], gsems[s])
      if not paired:
        pltpu.async_copy(zf_hbm.at[g1_v.at[s]], r1s[s], gsems[s])

    def gather_wait(s):
      pltpu.make_async_copy(zf_hbm.at[g0_v.at[s]], r0s[s], gsems[s]).wait()
      if not paired:
        pltpu.make_async_copy(zf_hbm.at[g1_v.at[s]], r1s[s], gsems[s]).wait()

    def interp_scatter(s):
      r0 = r0s[s]
      r1 = r0 if paired else r1s[s]
      boff = cout if paired else 0

      @plsc.parallel_loop(0, CN // LANES, 1, unroll=2)
      def interp(t):
        fg = f_v[s, pl.ds(t * LANES, LANES)]

        @plsc.parallel_loop(0, LANES // 4, 1, unroll=2)
        def inner(l4):
          for dl in range(4):
            l = l4 * 4 + dl
            fb = jnp.take_along_axis(
                fg, jnp.full((LANES,), l, jnp.int32), axis=0,
                mode="promise_in_bounds")
            e = t * LANES + l
            for j in range(jpr):
              a = r0[e, pl.ds(j * LANES, LANES)]
              b = r1[e, pl.ds(boff + j * LANES, LANES)]
              msg[e, pl.ds(j * LANES, LANES)] = a + fb * (b - a)

      pltpu.sync_copy(msg, acc.at[dst_v.at[s]], add=True)

    pltpu.sync_copy(src_hbm.at[pl.ds(ebase, CN)], src_v)
    pltpu.sync_copy(dst_hbm.at[pl.ds(ebase, CN)], dst_v.at[0])
    pltpu.sync_copy(ps_hbm.at[pl.ds(ebase, CN)], ps_v)
    idx_compute(0)
    gather_start(0)
    if nfull > 1:
      lin_start(1, 1)

    def pipe_step(u, s):
      @pl.when(u + 1 < nfull)
      def _():
        lin_wait(u + 1, 1 - s)
        idx_compute(1 - s)
        gather_start(1 - s)
      gather_wait(s)
      interp_scatter(s)
      @pl.when(u + 2 < nfull)
      def _():
        lin_start(u + 2, s)

    def pipe_pair(v, carry):
      pipe_step(2 * v, 0)
      pipe_step(2 * v + 1, 1)
      return carry

    lax.fori_loop(0, nfull // 2, pipe_pair, 0)

    if tail:
      tb = ebase + nfull * CN
      pltpu.sync_copy(src_hbm.at[pl.ds(tb, tail)], src_v.at[pl.ds(0, tail)])
      pltpu.sync_copy(dst_hbm.at[pl.ds(tb, tail)], dst_t)
      pltpu.sync_copy(ps_hbm.at[pl.ds(tb, tail)], ps_v.at[pl.ds(0, tail)])
      for j in range(tail // LANES):
        sl = pl.ds(j * LANES, LANES)
        v = ps_v[sl] * jnp.float32(K - 1)
        i0 = v.astype(jnp.int32)
        f_v[0, sl] = v - i0.astype(jnp.float32)
        row0 = src_v[sl] * K + i0
        g0_t[sl] = row0
        if not paired:
          g1_t[sl] = row0 + 1
      pltpu.async_copy(zf_hbm.at[g0_t], r0a.at[pl.ds(0, tail)], gsema).wait()
      if not paired:
        pltpu.async_copy(zf_hbm.at[g1_t], r1a.at[pl.ds(0, tail)], gsemb).wait()
      boff = cout if paired else 0
      r1 = r0a if paired else r1a
      for t in range(tail // LANES):
        fg = f_v[0, pl.ds(t * LANES, LANES)]
        for l in range(LANES):
          fb = jnp.take_along_axis(
              fg, jnp.full((LANES,), l, jnp.int32), axis=0,
              mode="promise_in_bounds")
          e = t * LANES + l
          for j in range(jpr):
            a = r0a[e, pl.ds(j * LANES, LANES)]
            b = r1[e, pl.ds(boff + j * LANES, LANES)]
            msg[e, pl.ds(j * LANES, LANES)] = a + fb * (b - a)
      pltpu.sync_copy(msg.at[pl.ds(0, tail)], acc.at[dst_t], add=True)

    plsc.subcore_barrier()
    pltpu.sync_copy(acc.at[pl.ds(sid * gpt, gpt)],
                    out_hbm.at[cid, pl.ds(sid * gpt, gpt)])
    if gtail:
      @pl.when(sid < gtail)
      def _():
        t0 = NS * gpt + sid * 8
        pltpu.sync_copy(acc.at[pl.ds(t0, 8)], out_hbm.at[cid, pl.ds(t0, 8)])

  return body


def _deg_kernel(N, E):
  chunks = E // C
  nfull = chunks // NW
  rem = chunks - nfull * NW
  gpt, gtail = _row_chunks(N)
  mesh = plsc.VectorSubcoreMesh(core_axis_name="c", subcore_axis_name="s")

  @functools.partial(
      pl.kernel,
      mesh=mesh,
      out_type=jax.ShapeDtypeStruct((2, N, G), jnp.float32),
      scratch_types=[
          pltpu.VMEM((C,), jnp.int32),
          pltpu.VMEM((C, G), jnp.float32),
          pltpu.VMEM_SHARED((N, G), jnp.float32),
      ],
  )
  def body(dst_hbm, zeros_hbm, ones_hbm, out_hbm, dst_v, ones_v, acc):
    cid = lax.axis_index("c")
    sid = lax.axis_index("s")
    wid = cid * NS + sid

    pltpu.sync_copy(ones_hbm, ones_v)
    pltpu.sync_copy(zeros_hbm.at[pl.ds(sid * gpt, gpt)],
                    acc.at[pl.ds(sid * gpt, gpt)])
    if gtail:
      @pl.when(sid < gtail)
      def _():
        t0 = NS * gpt + sid * 8
        pltpu.sync_copy(zeros_hbm.at[pl.ds(t0, 8)], acc.at[pl.ds(t0, 8)])
    plsc.subcore_barrier()

    def do_chunk(c_idx):
      pltpu.sync_copy(dst_hbm.at[pl.ds(c_idx * C, C)], dst_v)
      pltpu.sync_copy(ones_v, acc.at[dst_v], add=True)

    def chunk_loop(t, carry):
      do_chunk(wid + NW * t)
      return carry

    lax.fori_loop(0, nfull, chunk_loop, 0)
    if rem:
      @pl.when(wid < rem)
      def _():
        do_chunk(NW * nfull + wid)

    plsc.subcore_barrier()
    pltpu.sync_copy(acc.at[pl.ds(sid * gpt, gpt)],
                    out_hbm.at[cid, pl.ds(sid * gpt, gpt)])
    if gtail:
      @pl.when(sid < gtail)
      def _():
        t0 = NS * gpt + sid * 8
        pltpu.sync_copy(acc.at[pl.ds(t0, 8)], out_hbm.at[cid, pl.ds(t0, 8)])

  return body


_BR = 1000


def _pad_table(z, K, cout):
  br = z.shape[0]
  if cout == G:
    return z
  zslot = jnp.zeros((br, cout), jnp.float32)
  pieces = []
  for k in range(K):
    pieces.append(z[:, k * cout:(k + 1) * cout])
    pieces.append(z[:, (k + 1) * cout:(k + 2) * cout] if k + 1 < K
                  else zslot)
  return jnp.concatenate(pieces, axis=1)


def _table_width(K, cout):
  return K * G if cout == G else K * 2 * cout


def _mm0_body(x_ref, w_ref, o1_ref, o2_ref, *, K, cout):
  kc = K * cout
  z = jnp.dot(x_ref[...], w_ref[...], preferred_element_type=jnp.float32)
  o1_ref[...] = _pad_table(z[:, :kc], K, cout)
  o2_ref[...] = z[:, kc:]


def _epilogue(p_ref, dp_ref, rt_ref, b_ref, cp, elu, dlane):
  d = dp_ref[0, :, dlane:dlane + 1] + dp_ref[1, :, dlane:dlane + 1]
  inv = 1.0 / jnp.maximum(d, 1.0)
  h = (p_ref[0, :, :cp] + p_ref[1, :, :cp]) * inv + rt_ref[...] + b_ref[...]
  if elu:
    h = jnp.where(h > 0, h, jnp.exp(jnp.minimum(h, 0.0)) - 1.0)
  return h


def _mm_fused_body(p_ref, dp_ref, rt_ref, b_ref, w_ref, o1_ref, o2_ref,
                   *, cp, K, cout, elu, dlane):
  kc = K * cout
  h = _epilogue(p_ref, dp_ref, rt_ref, b_ref, cp, elu, dlane)
  z = jnp.dot(h, w_ref[...], preferred_element_type=jnp.float32)
  o1_ref[...] = _pad_table(z[:, :kc], K, cout)
  o2_ref[...] = z[:, kc:]


def _final_body(p_ref, dp_ref, rt_ref, b_ref, o_ref, *, cp, dlane):
  h = _epilogue(p_ref, dp_ref, rt_ref, b_ref, cp, False, dlane)
  m = jnp.max(h, axis=1, keepdims=True)
  l = h - m
  s = jnp.sum(jnp.exp(l), axis=1, keepdims=True)
  o_ref[...] = l - jnp.log(s)


def _mm0(N, cin, K, cout):
  cols = (K + 1) * cout
  return pl.pallas_call(
      functools.partial(_mm0_body, K=K, cout=cout),
      grid=(N // _BR,),
      in_specs=[
          pl.BlockSpec((_BR, cin), lambda i: (i, 0)),
          pl.BlockSpec((cin, cols), lambda i: (0, 0)),
      ],
      out_specs=[
          pl.BlockSpec((_BR, _table_width(K, cout)), lambda i: (i, 0)),
          pl.BlockSpec((_BR, cout), lambda i: (i, 0)),
      ],
      out_shape=[
          jax.ShapeDtypeStruct((N, _table_width(K, cout)), jnp.float32),
          jax.ShapeDtypeStruct((N, cout), jnp.float32),
      ],
  )


def _mm_fused(N, cp, K, cout, elu, dlane, pw, dw):
  cols = (K + 1) * cout
  return pl.pallas_call(
      functools.partial(_mm_fused_body, cp=cp, K=K, cout=cout, elu=elu,
                        dlane=dlane),
      grid=(N // _BR,),
      in_specs=[
          pl.BlockSpec((2, _BR, pw), lambda i: (0, i, 0)),
          pl.BlockSpec((2, _BR, dw), lambda i: (0, i, 0)),
          pl.BlockSpec((_BR, cp), lambda i: (i, 0)),
          pl.BlockSpec((1, cp), lambda i: (0, 0)),
          pl.BlockSpec((cp, cols), lambda i: (0, 0)),
      ],
      out_specs=[
          pl.BlockSpec((_BR, _table_width(K, cout)), lambda i: (i, 0)),
          pl.BlockSpec((_BR, cout), lambda i: (i, 0)),
      ],
      out_shape=[
          jax.ShapeDtypeStruct((N, _table_width(K, cout)), jnp.float32),
          jax.ShapeDtypeStruct((N, cout), jnp.float32),
      ],
  )


def _final(N, cp, dlane, pw, dw):
  return pl.pallas_call(
      functools.partial(_final_body, cp=cp, dlane=dlane),
      grid=(N // _BR,),
      in_specs=[
          pl.BlockSpec((2, _BR, pw), lambda i: (0, i, 0)),
          pl.BlockSpec((2, _BR, dw), lambda i: (0, i, 0)),
          pl.BlockSpec((_BR, cp), lambda i: (i, 0)),
          pl.BlockSpec((1, cp), lambda i: (0, 0)),
      ],
      out_specs=pl.BlockSpec((_BR, cp), lambda i: (i, 0)),
      out_shape=jax.ShapeDtypeStruct((N, cp), jnp.float32),
  )


@jax.jit
def kernel(x, edge_index, pseudo, params):
  N = x.shape[0]
  E = edge_index.shape[1]
  src = edge_index[0]
  dst = edge_index[1]
  ps = pseudo[:, 0]

  layer_dims = [(p["weight"].shape[1], p["weight"].shape[2], p["weight"].shape[0])
                for p in params]
  wcats = []
  for p, (cin, cout, K) in zip(params, layer_dims):
    wflat = jnp.transpose(p["weight"], (1, 0, 2)).reshape(cin, K * cout)
    wcats.append(jnp.concatenate([wflat, p["root"]], axis=1))

  cin0, cout0, k0 = layer_dims[0]
  z, rt = _mm0(N, cin0, k0, cout0)(x, wcats[0])

  deg_parts = None
  dlane = layer_dims[0][1]
  dw = dlane + LANES
  for li, (cin, cout, K) in enumerate(layer_dims):
    paired = 2 * cout <= G
    wa = dw if li == 0 else (cout if paired else G)
    w = 2 * cout if paired else G
    parts = _edge_kernel(N, E, K, cout, with_deg=(li == 0))(
        z.reshape(N * K, w), src, dst, ps, jnp.zeros((N, wa), jnp.float32))
    if li == 0:
      deg_parts = parts
    bias = params[li]["bias"].reshape(1, cout)
    if li + 1 < len(layer_dims):
      cin_n, cout_n, k_n = layer_dims[li + 1]
      z, rt = _mm_fused(N, cout, k_n, cout_n, elu=li in (0, 2, 4),
                        dlane=dlane, pw=wa, dw=dw)(
          parts, deg_parts, rt, bias, wcats[li + 1])
    else:
      return _final(N, cout, dlane, pw=wa, dw=dw)(parts, deg_parts, rt, bias)
